# initial kernel scaffold (unmeasured)
import jax
import jax.numpy as jnp
from jax import lax
from jax.experimental import pallas as pl
from jax.experimental.pallas import tpu as pltpu


def kernel(
    x,
):
    def body(*refs):
        pass

    out_shape = jax.ShapeDtypeStruct(..., jnp.float32)
    return pl.pallas_call(body, out_shape=out_shape)(...)


# baseline (device time: 13377 ns/iter reference)
import jax
import jax.numpy as jnp
from jax import lax
from jax.experimental import pallas as pl
from jax.experimental.pallas import tpu as pltpu

N_DEV = 4


def kernel(x):
    m, n = x.shape

    def body(x_ref, out_ref, recv_ref, send_sems, recv_sems):
        p = lax.axis_index("i")
        a = p ^ 1
        b = (N_DEV - 1) - p

        barrier_sem = pltpu.get_barrier_semaphore()
        for nbr in [a, b]:
            pl.semaphore_signal(
                barrier_sem, inc=1,
                device_id=(nbr,), device_id_type=pl.DeviceIdType.MESH,
            )
        pl.semaphore_wait(barrier_sem, 2)

        rdma1 = pltpu.make_async_remote_copy(
            src_ref=x_ref,
            dst_ref=recv_ref.at[0],
            send_sem=send_sems.at[0],
            recv_sem=recv_sems.at[0],
            device_id=(a,),
            device_id_type=pl.DeviceIdType.MESH,
        )
        rdma1.start()
        rdma1.wait()
        out_ref[...] = x_ref[...] + recv_ref[0]

        rdma2 = pltpu.make_async_remote_copy(
            src_ref=out_ref,
            dst_ref=recv_ref.at[1],
            send_sem=send_sems.at[1],
            recv_sem=recv_sems.at[1],
            device_id=(b,),
            device_id_type=pl.DeviceIdType.MESH,
        )
        rdma2.start()
        rdma2.wait()
        out_ref[...] = out_ref[...] + recv_ref[1]

    return pl.pallas_call(
        body,
        out_shape=jax.ShapeDtypeStruct((m, n), x.dtype),
        in_specs=[pl.BlockSpec(memory_space=pltpu.VMEM)],
        out_specs=pl.BlockSpec(memory_space=pltpu.VMEM),
        scratch_shapes=[
            pltpu.VMEM((2, m, n), x.dtype),
            pltpu.SemaphoreType.DMA((2,)),
            pltpu.SemaphoreType.DMA((2,)),
        ],
        compiler_params=pltpu.CompilerParams(collective_id=0),
    )(x)


# device time: 10637 ns/iter; 1.2576x vs baseline; 1.2576x over previous
import jax
import jax.numpy as jnp
from jax import lax
from jax.experimental import pallas as pl
from jax.experimental.pallas import tpu as pltpu

N_DEV = 4


def kernel(x):
    m, n = x.shape
    hm = m // 2

    def body(x_ref, out_ref, recv_ref, send_sems, recv_sems):
        p = lax.axis_index("i")
        a = p ^ 1
        b = (N_DEV - 1) - p

        barrier_sem = pltpu.get_barrier_semaphore()
        for nbr in [a, b]:
            pl.semaphore_signal(
                barrier_sem, inc=1,
                device_id=(nbr,), device_id_type=pl.DeviceIdType.MESH,
            )
        pl.semaphore_wait(barrier_sem, 2)

        def exchange(src, stream, phase, dev):
            slot = stream * 2 + phase
            return pltpu.make_async_remote_copy(
                src_ref=src,
                dst_ref=recv_ref.at[slot],
                send_sem=send_sems.at[slot],
                recv_sem=recv_sems.at[slot],
                device_id=(dev,),
                device_id_type=pl.DeviceIdType.MESH,
            )

        rows_a = pl.ds(0, hm)
        rows_b = pl.ds(hm, hm)

        rdma_a1 = exchange(x_ref.at[rows_a], 0, 0, a)
        rdma_b1 = exchange(x_ref.at[rows_b], 1, 0, b)
        rdma_a1.start()
        rdma_b1.start()

        rdma_a1.wait()
        out_ref[rows_a, :] = x_ref[rows_a, :] + recv_ref[0]
        rdma_a2 = exchange(out_ref.at[rows_a], 0, 1, b)
        rdma_a2.start()

        rdma_b1.wait()
        out_ref[rows_b, :] = x_ref[rows_b, :] + recv_ref[2]
        rdma_b2 = exchange(out_ref.at[rows_b], 1, 1, a)
        rdma_b2.start()

        rdma_a2.wait()
        out_ref[rows_a, :] = out_ref[rows_a, :] + recv_ref[1]
        rdma_b2.wait()
        out_ref[rows_b, :] = out_ref[rows_b, :] + recv_ref[3]

    return pl.pallas_call(
        body,
        out_shape=jax.ShapeDtypeStruct((m, n), x.dtype),
        in_specs=[pl.BlockSpec(memory_space=pltpu.VMEM)],
        out_specs=pl.BlockSpec(memory_space=pltpu.VMEM),
        scratch_shapes=[
            pltpu.VMEM((4, hm, n), x.dtype),
            pltpu.SemaphoreType.DMA((4,)),
            pltpu.SemaphoreType.DMA((4,)),
        ],
        compiler_params=pltpu.CompilerParams(collective_id=0),
    )(x)


# device time: 9291 ns/iter; 1.4398x vs baseline; 1.1449x over previous
import jax
import jax.numpy as jnp
from jax import lax
from jax.experimental import pallas as pl
from jax.experimental.pallas import tpu as pltpu

N_DEV = 4
N_CHUNKS = 4


def kernel(x):
    m, n = x.shape
    qm = m // N_CHUNKS

    def body(x_ref, out_ref, recv_ref, send_sems, recv_sems):
        p = lax.axis_index("i")
        a = p ^ 1
        b = (N_DEV - 1) - p

        barrier_sem = pltpu.get_barrier_semaphore()
        for nbr in [a, b]:
            pl.semaphore_signal(
                barrier_sem, inc=1,
                device_id=(nbr,), device_id_type=pl.DeviceIdType.MESH,
            )
        pl.semaphore_wait(barrier_sem, 2)

        def exchange(src, slot, dev):
            return pltpu.make_async_remote_copy(
                src_ref=src,
                dst_ref=recv_ref.at[slot],
                send_sem=send_sems.at[slot],
                recv_sem=recv_sems.at[slot],
                device_id=(dev,),
                device_id_type=pl.DeviceIdType.MESH,
            )

        ph1_dev = [a, a, b, b]
        ph2_dev = [b, b, a, a]
        rows = [pl.ds(i * qm, qm) for i in range(N_CHUNKS)]
        order = [0, 2, 1, 3]

        ph1 = [exchange(x_ref.at[rows[i]], i, ph1_dev[i]) for i in range(N_CHUNKS)]
        for i in range(N_CHUNKS):
            ph1[i].start()

        ph2 = [None] * N_CHUNKS
        for i in order:
            ph1[i].wait_recv()
            out_ref[rows[i], :] = x_ref[rows[i], :] + recv_ref[i]
            ph2[i] = exchange(out_ref.at[rows[i]], N_CHUNKS + i, ph2_dev[i])
            ph2[i].start()

        for i in order:
            ph2[i].wait()
            out_ref[rows[i], :] = out_ref[rows[i], :] + recv_ref[N_CHUNKS + i]

        for i in range(N_CHUNKS):
            ph1[i].wait_send()

    return pl.pallas_call(
        body,
        out_shape=jax.ShapeDtypeStruct((m, n), x.dtype),
        in_specs=[pl.BlockSpec(memory_space=pltpu.VMEM)],
        out_specs=pl.BlockSpec(memory_space=pltpu.VMEM),
        scratch_shapes=[
            pltpu.VMEM((2 * N_CHUNKS, qm, n), x.dtype),
            pltpu.SemaphoreType.DMA((2 * N_CHUNKS,)),
            pltpu.SemaphoreType.DMA((2 * N_CHUNKS,)),
        ],
        compiler_params=pltpu.CompilerParams(collective_id=0),
    )(x)
